# baseline (device time: 80429 ns/iter reference)
import jax
import jax.numpy as jnp
from jax import lax
from jax.experimental import pallas as pl
from jax.experimental.pallas import tpu as pltpu

N_DEV = 32
BLK = 128
KC = 512


def kernel(x, w_mat):
    m, _kblk = x.shape
    k, n = w_mat.shape
    n_chunks = k // KC

    def body(x_ref, w_ref, out_ref, xg_ref, wbuf_ref, send_sems, recv_sems,
             w_sems):
        me = lax.axis_index("i")

        def w_copy(c, slot):
            return pltpu.make_async_copy(
                w_ref.at[pl.ds(c * KC, KC), :],
                wbuf_ref.at[slot],
                w_sems.at[slot],
            )

        w_copy(0, 0).start()
        w_copy(1, 1).start()

        xg_ref[:, pl.ds(me * BLK, BLK)] = x_ref[pl.ds(me * BLK, BLK), :]

        for o in range(1, N_DEV):
            d = lax.rem(me + o, N_DEV)
            pltpu.make_async_remote_copy(
                src_ref=x_ref.at[pl.ds(d * BLK, BLK), :],
                dst_ref=xg_ref.at[:, pl.ds(me * BLK, BLK)],
                send_sem=send_sems.at[o],
                recv_sem=recv_sems.at[me],
                device_id=(d,),
                device_id_type=pl.DeviceIdType.MESH,
            ).start()

        for o in range(1, N_DEV):
            j = lax.rem(me + o, N_DEV)
            pltpu.make_async_remote_copy(
                src_ref=x_ref.at[pl.ds(0, BLK), :],
                dst_ref=xg_ref.at[:, pl.ds(j * BLK, BLK)],
                send_sem=send_sems.at[o],
                recv_sem=recv_sems.at[j],
                device_id=(me,),
                device_id_type=pl.DeviceIdType.MESH,
            ).wait_recv()

        for c in range(n_chunks):
            slot = c % 2
            w_copy(c, slot).wait()
            acc = jnp.dot(
                xg_ref[:, pl.ds(c * KC, KC)], wbuf_ref[slot],
                preferred_element_type=jnp.float32,
            )
            if c == 0:
                out_ref[...] = acc
            else:
                out_ref[...] = out_ref[...] + acc
            if c + 2 < n_chunks:
                w_copy(c + 2, slot).start()

        for o in range(1, N_DEV):
            d = lax.rem(me + o, N_DEV)
            pltpu.make_async_remote_copy(
                src_ref=x_ref.at[pl.ds(d * BLK, BLK), :],
                dst_ref=xg_ref.at[:, pl.ds(me * BLK, BLK)],
                send_sem=send_sems.at[o],
                recv_sem=recv_sems.at[me],
                device_id=(d,),
                device_id_type=pl.DeviceIdType.MESH,
            ).wait_send()

    return pl.pallas_call(
        body,
        out_shape=jax.ShapeDtypeStruct((BLK, n), jnp.float32),
        in_specs=[
            pl.BlockSpec(memory_space=pltpu.VMEM),
            pl.BlockSpec(memory_space=pl.ANY),
        ],
        out_specs=pl.BlockSpec(memory_space=pltpu.VMEM),
        scratch_shapes=[
            pltpu.VMEM((BLK, m), jnp.float32),
            pltpu.VMEM((2, KC, n), jnp.float32),
            pltpu.SemaphoreType.DMA((N_DEV,)),
            pltpu.SemaphoreType.DMA((N_DEV,)),
            pltpu.SemaphoreType.DMA((2,)),
        ],
        compiler_params=pltpu.CompilerParams(
            vmem_limit_bytes=56 * 1024 * 1024,
        ),
    )(x, w_mat)


# device time: 61460 ns/iter; 1.3086x vs baseline; 1.3086x over previous
import jax
import jax.numpy as jnp
from jax import lax
from jax.experimental import pallas as pl
from jax.experimental.pallas import tpu as pltpu

N_DEV = 32
BLK = 128
KC = 256
SRC_PER_CHUNK = KC // BLK


def kernel(x, w_mat):
    m, _kblk = x.shape
    k, n = w_mat.shape
    n_chunks = k // KC

    def body(x_ref, w_ref, out_ref, xbf_ref, xg_ref, wbuf_ref, wbf_ref,
             send_sems, recv_sems, w_sems):
        me = lax.axis_index("i")
        c0 = lax.div(me, SRC_PER_CHUNK)

        def chunk_id(c):
            return lax.rem(c0 + c, n_chunks)

        def w_copy(c, slot):
            return pltpu.make_async_copy(
                w_ref.at[pl.ds(chunk_id(c) * KC, KC), :],
                wbuf_ref.at[slot],
                w_sems.at[slot],
            )

        w_copy(0, 0).start()
        w_copy(1, 1).start()

        xbf_ref[...] = x_ref[...].astype(jnp.bfloat16)

        xg_ref[:, pl.ds(me * BLK, BLK)] = xbf_ref[pl.ds(me * BLK, BLK), :]

        for o in range(1, N_DEV):
            d = lax.rem(me - o + N_DEV, N_DEV)
            pltpu.make_async_remote_copy(
                src_ref=xbf_ref.at[pl.ds(d * BLK, BLK), :],
                dst_ref=xg_ref.at[:, pl.ds(me * BLK, BLK)],
                send_sem=send_sems.at[o],
                recv_sem=recv_sems.at[me],
                device_id=(d,),
                device_id_type=pl.DeviceIdType.MESH,
            ).start()

        for c in range(n_chunks):
            slot = c % 2
            for s in range(SRC_PER_CHUNK):
                j = chunk_id(c) * SRC_PER_CHUNK + s
                skip = j == me

                @pl.when(jnp.logical_not(skip))
                def _(j=j):
                    pltpu.make_async_remote_copy(
                        src_ref=xbf_ref.at[pl.ds(0, BLK), :],
                        dst_ref=xg_ref.at[:, pl.ds(j * BLK, BLK)],
                        send_sem=send_sems.at[0],
                        recv_sem=recv_sems.at[j],
                        device_id=(me,),
                        device_id_type=pl.DeviceIdType.MESH,
                    ).wait_recv()

            w_copy(c, slot).wait()
            wbf_ref[slot] = wbuf_ref[slot].astype(jnp.bfloat16)
            if c + 2 < n_chunks:
                w_copy(c + 2, slot).start()
            acc = jnp.dot(
                xg_ref[:, pl.ds(chunk_id(c) * KC, KC)], wbf_ref[slot],
                preferred_element_type=jnp.float32,
            )
            if c == 0:
                out_ref[...] = acc
            else:
                out_ref[...] = out_ref[...] + acc

        for o in range(1, N_DEV):
            d = lax.rem(me - o + N_DEV, N_DEV)
            pltpu.make_async_remote_copy(
                src_ref=xbf_ref.at[pl.ds(d * BLK, BLK), :],
                dst_ref=xg_ref.at[:, pl.ds(me * BLK, BLK)],
                send_sem=send_sems.at[o],
                recv_sem=recv_sems.at[me],
                device_id=(d,),
                device_id_type=pl.DeviceIdType.MESH,
            ).wait_send()

    return pl.pallas_call(
        body,
        out_shape=jax.ShapeDtypeStruct((BLK, n), jnp.float32),
        in_specs=[
            pl.BlockSpec(memory_space=pltpu.VMEM),
            pl.BlockSpec(memory_space=pl.ANY),
        ],
        out_specs=pl.BlockSpec(memory_space=pltpu.VMEM),
        scratch_shapes=[
            pltpu.VMEM((m, BLK), jnp.bfloat16),
            pltpu.VMEM((BLK, m), jnp.bfloat16),
            pltpu.VMEM((2, KC, n), jnp.float32),
            pltpu.VMEM((2, KC, n), jnp.bfloat16),
            pltpu.SemaphoreType.DMA((N_DEV,)),
            pltpu.SemaphoreType.DMA((N_DEV,)),
            pltpu.SemaphoreType.DMA((2,)),
        ],
        compiler_params=pltpu.CompilerParams(
            vmem_limit_bytes=48 * 1024 * 1024,
        ),
    )(x, w_mat)
